# baseline (speedup 1.0000x reference)
import functools

import jax
import jax.numpy as jnp
from jax import lax
from jax.experimental import pallas as pl
from jax.experimental.pallas import tpu as pltpu
from jax.experimental.pallas import tpu_sc as plsc

N = 16384
E = 262144
D = 256
B = 32
EPS = 1e-5

_SENT = -3.0e38
_BM = 512


def _mm_body(x_ref, w_ref, b_ref, *out_refs, relu, stats):
    y = jnp.dot(x_ref[...], w_ref[...], preferred_element_type=jnp.float32,
                precision=lax.Precision.HIGHEST)
    y = y + b_ref[...]
    if relu:
        y = jnp.maximum(y, 0.0)
    out_refs[0][...] = y
    if stats:
        s_ref, q_ref = out_refs[1], out_refs[2]

        @pl.when(pl.program_id(0) == 0)
        def _():
            s_ref[...] = jnp.zeros_like(s_ref)
            q_ref[...] = jnp.zeros_like(q_ref)

        s_ref[...] += jnp.sum(y, axis=0, keepdims=True)
        q_ref[...] += jnp.sum(y * y, axis=0, keepdims=True)


def _mm(x, w, bias, relu=False, stats=False):
    n, di = x.shape
    do = w.shape[1]
    outs = [jax.ShapeDtypeStruct((n, do), jnp.float32)]
    out_specs = [pl.BlockSpec((_BM, do), lambda i: (i, 0))]
    if stats:
        outs += [jax.ShapeDtypeStruct((1, do), jnp.float32)] * 2
        out_specs += [pl.BlockSpec((1, do), lambda i: (0, 0))] * 2
    return pl.pallas_call(
        functools.partial(_mm_body, relu=relu, stats=stats),
        grid=(n // _BM,),
        in_specs=[
            pl.BlockSpec((_BM, di), lambda i: (i, 0)),
            pl.BlockSpec((di, do), lambda i: (0, 0)),
            pl.BlockSpec((1, do), lambda i: (0, 0)),
        ],
        out_specs=out_specs,
        out_shape=outs,
    )(x, w, bias.reshape(1, -1))


def _mm_ab_body(x_ref, w_ref, b_ref, a_ref, bo_ref):
    y = jnp.dot(x_ref[...], w_ref[...], preferred_element_type=jnp.float32,
                precision=lax.Precision.HIGHEST)
    y = y + b_ref[...]
    a_ref[...] = y[:, :D]
    bo_ref[...] = y[:, D:]


def _mm_ab(x, w2, bias2):
    n = x.shape[0]
    return pl.pallas_call(
        _mm_ab_body,
        grid=(n // _BM,),
        in_specs=[
            pl.BlockSpec((_BM, D), lambda i: (i, 0)),
            pl.BlockSpec((D, 2 * D), lambda i: (0, 0)),
            pl.BlockSpec((1, 2 * D), lambda i: (0, 0)),
        ],
        out_specs=[
            pl.BlockSpec((_BM, D), lambda i: (i, 0)),
            pl.BlockSpec((_BM, D), lambda i: (i, 0)),
        ],
        out_shape=[jax.ShapeDtypeStruct((n, D), jnp.float32)] * 2,
    )(x, w2, bias2.reshape(1, -1))


def _combine_body(seg_ref, bv_ref, h_ref, u_ref, s_ref, q_ref):
    seg = seg_ref[...]
    agg = jnp.where(seg < -1e38, 0.0, seg + bv_ref[...])
    u = jnp.maximum(agg + h_ref[...], 0.0)
    u_ref[...] = u

    @pl.when(pl.program_id(0) == 0)
    def _():
        s_ref[...] = jnp.zeros_like(s_ref)
        q_ref[...] = jnp.zeros_like(q_ref)

    s_ref[...] += jnp.sum(u, axis=0, keepdims=True)
    q_ref[...] += jnp.sum(u * u, axis=0, keepdims=True)


def _combine(seg, bvec, h):
    n = h.shape[0]
    return pl.pallas_call(
        _combine_body,
        grid=(n // _BM,),
        in_specs=[pl.BlockSpec((_BM, D), lambda i: (i, 0))] * 3,
        out_specs=[
            pl.BlockSpec((_BM, D), lambda i: (i, 0)),
            pl.BlockSpec((1, D), lambda i: (0, 0)),
            pl.BlockSpec((1, D), lambda i: (0, 0)),
        ],
        out_shape=[
            jax.ShapeDtypeStruct((n, D), jnp.float32),
            jax.ShapeDtypeStruct((1, D), jnp.float32),
            jax.ShapeDtypeStruct((1, D), jnp.float32),
        ],
    )(seg, bvec, h)


def _bn_body(y_ref, s_ref, q_ref, g_ref, b_ref, o_ref, *, n_rows):
    mu = s_ref[...] / n_rows
    var = q_ref[...] / n_rows - mu * mu
    sc = g_ref[...] * lax.rsqrt(var + EPS)
    sh = b_ref[...] - mu * sc
    o_ref[...] = y_ref[...] * sc + sh


def _bn(y, s, q, g, b):
    n, do = y.shape
    return pl.pallas_call(
        functools.partial(_bn_body, n_rows=float(n)),
        grid=(n // _BM,),
        in_specs=[
            pl.BlockSpec((_BM, do), lambda i: (i, 0)),
            pl.BlockSpec((1, do), lambda i: (0, 0)),
            pl.BlockSpec((1, do), lambda i: (0, 0)),
            pl.BlockSpec((1, do), lambda i: (0, 0)),
            pl.BlockSpec((1, do), lambda i: (0, 0)),
        ],
        out_specs=pl.BlockSpec((_BM, do), lambda i: (i, 0)),
        out_shape=jax.ShapeDtypeStruct((n, do), jnp.float32),
    )(y, s.reshape(1, -1), q.reshape(1, -1), g.reshape(1, -1), b.reshape(1, -1))


_NW = 32
_RANGE = N // _NW
_HALF = _RANGE // 2
_NB = 2 * _NW
_SCAN = 8192
_GRAN = 1024
_FLUSH = 2048
_RING = 4096
_G = 32
_NCH = _FLUSH // _G
_NBUF = 4


def _bucketize(src, dst):
    mesh = plsc.VectorSubcoreMesh(core_axis_name="c", subcore_axis_name="s")

    @functools.partial(
        pl.kernel,
        mesh=mesh,
        compiler_params=pltpu.CompilerParams(needs_layout_passes=False),
        out_type=(
            jax.ShapeDtypeStruct((_NB, E), jnp.int32),
            jax.ShapeDtypeStruct((_NB, E), jnp.int32),
            jax.ShapeDtypeStruct((_NB, 16), jnp.int32),
        ),
        scratch_types=[
            pltpu.VMEM((_SCAN,), jnp.int32),
            pltpu.VMEM((_SCAN,), jnp.int32),
            pltpu.VMEM((_RING,), jnp.int32),
            pltpu.VMEM((_RING,), jnp.int32),
            pltpu.VMEM((_RING,), jnp.int32),
            pltpu.VMEM((_RING,), jnp.int32),
            pltpu.VMEM((16,), jnp.int32),
        ],
    )
    def k(src_hbm, dst_hbm, bsrc_hbm, bdl_hbm, cnt_hbm,
          sbuf, dbuf, rs0, rd0, rs1, rd1, cntv):
        wid = lax.axis_index("s") * 2 + lax.axis_index("c")
        kb0 = wid * 2
        lo = wid * _RANGE

        def prefill(i, _):
            z = jnp.zeros((16,), jnp.int32)
            hv = jnp.full((16,), _HALF, jnp.int32)
            rs0[pl.ds(i * 16, 16)] = z
            rs1[pl.ds(i * 16, 16)] = z
            rd0[pl.ds(i * 16, 16)] = hv
            rd1[pl.ds(i * 16, 16)] = hv
            return 0

        lax.fori_loop(0, _RING // 16, prefill, 0)

        def flush(f, pend, rs, rd, kb, final):
            def cond(f):
                lim = (f + 1) * _FLUSH if not final else f * _FLUSH
                return jnp.any(pend > lim - (0 if final else 1))

            def body(f):
                blk = lax.rem(f, 2) * _FLUSH
                pltpu.sync_copy(rs.at[pl.ds(blk, _FLUSH)],
                                bsrc_hbm.at[kb, pl.ds(f * _FLUSH, _FLUSH)])
                pltpu.sync_copy(rd.at[pl.ds(blk, _FLUSH)],
                                bdl_hbm.at[kb, pl.ds(f * _FLUSH, _FLUSH)])
                return f + 1

            return lax.while_loop(cond, body, f)

        def outer(oc, carry):
            base = oc * _SCAN
            pltpu.sync_copy(src_hbm.at[pl.ds(base, _SCAN)], sbuf)
            pltpu.sync_copy(dst_hbm.at[pl.ds(base, _SCAN)], dbuf)

            def gran(gi, carry):
                f0, f1, p0, p1 = carry

                def scan16(j, carry):
                    p0, p1 = carry
                    off = gi * _GRAN + j * 16
                    sv = sbuf[pl.ds(off, 16)]
                    dv = dbuf[pl.ds(off, 16)]
                    dl = dv - lo
                    m0 = (dl >= 0) & (dl < _HALF)
                    m1 = (dl >= _HALF) & (dl < _RANGE)
                    cs0 = plsc.cumsum(m0.astype(jnp.int32))
                    pos0 = (p0 + cs0 - 1) & (_RING - 1)
                    plsc.store_scatter(rs0, [pos0], sv, mask=m0)
                    plsc.store_scatter(rd0, [pos0], dl, mask=m0)
                    cs1 = plsc.cumsum(m1.astype(jnp.int32))
                    pos1 = (p1 + cs1 - 1) & (_RING - 1)
                    plsc.store_scatter(rs1, [pos1], sv, mask=m1)
                    plsc.store_scatter(rd1, [pos1], dl - _HALF, mask=m1)
                    return (p0 + plsc.all_reduce_population_count(m0),
                            p1 + plsc.all_reduce_population_count(m1))

                p0, p1 = lax.fori_loop(0, _GRAN // 16, scan16, (p0, p1))
                f0 = flush(f0, p0, rs0, rd0, kb0, False)
                f1 = flush(f1, p1, rs1, rd1, kb0 + 1, False)
                return (f0, f1, p0, p1)

            return lax.fori_loop(0, _SCAN // _GRAN, gran, carry)

        z16 = jnp.zeros((16,), jnp.int32)
        f0, f1, p0, p1 = lax.fori_loop(
            0, E // _SCAN, outer, (jnp.int32(0), jnp.int32(0), z16, z16))
        f0 = flush(f0, p0, rs0, rd0, kb0, True)
        f1 = flush(f1, p1, rs1, rd1, kb0 + 1, True)
        cntv[pl.ds(0, 16)] = p0
        pltpu.sync_copy(cntv, cnt_hbm.at[kb0])
        cntv[pl.ds(0, 16)] = p1
        pltpu.sync_copy(cntv, cnt_hbm.at[kb0 + 1])

    return k(src, dst)


def _segmax(a, bsrc, bdl, counts):
    mesh = plsc.VectorSubcoreMesh(core_axis_name="c", subcore_axis_name="s")

    @functools.partial(
        pl.kernel,
        mesh=mesh,
        compiler_params=pltpu.CompilerParams(needs_layout_passes=False),
        out_type=jax.ShapeDtypeStruct((N, D), jnp.float32),
        scratch_types=[
            pltpu.VMEM((_HALF + 1, D), jnp.float32),
            pltpu.VMEM((_FLUSH,), jnp.int32),
            pltpu.VMEM((_FLUSH,), jnp.int32),
            pltpu.VMEM((_NBUF, _G, D), jnp.float32),
            pltpu.VMEM((16,), jnp.int32),
            pltpu.SemaphoreType.DMA,
        ],
    )
    def k(a_hbm, bsrc_hbm, bdl_hbm, cnt_hbm, seg_hbm,
          acc, psrc, pdst, rows, cntv, sem):
        wid = lax.axis_index("s") * 2 + lax.axis_index("c")
        lanes = jnp.arange(16, dtype=jnp.int32)

        for bi in range(2):
            kb = wid * 2 + bi
            lo = wid * _RANGE + bi * _HALF

            def initrow(i, _):
                for c in range(D // 16):
                    acc[i, pl.ds(c * 16, 16)] = jnp.full((16,), _SENT, jnp.float32)
                return 0

            lax.fori_loop(0, _HALF + 1, initrow, 0)

            pltpu.sync_copy(cnt_hbm.at[kb], cntv)
            cnt = cntv[pl.ds(0, 16)]

            def gather(g, buf):
                return pltpu.async_copy(
                    a_hbm.at[psrc.at[pl.ds(g * _G, _G)]], rows.at[buf], sem)

            def accum(g, buf):
                def row1(j):
                    jh = (j // 16) * 16
                    jm = j - jh
                    dchunk = pdst[pl.ds(g * _G + jh, 16)]
                    jvec = jnp.broadcast_to(jm, (16, 1)).astype(jnp.int32)
                    dlb = lax.gather(
                        dchunk, jvec,
                        lax.GatherDimensionNumbers(
                            offset_dims=(), collapsed_slice_dims=(0,),
                            start_index_map=(0,)),
                        (1,), mode=lax.GatherScatterMode.PROMISE_IN_BOUNDS)
                    for c in range(D // 16):
                        colidx = c * 16 + lanes
                        gv = rows[buf, j, pl.ds(c * 16, 16)]
                        av = plsc.load_gather(acc, [dlb, colidx])
                        plsc.store_scatter(
                            acc, [dlb, colidx], jnp.maximum(av, gv))

                def rowj(i, _):
                    for u in range(4):
                        row1(i * 4 + u)
                    return 0

                lax.fori_loop(0, _G // 4, rowj, 0)

            def cond(fb):
                return jnp.any(cnt > fb * _FLUSH)

            def body(fb):
                rem = cnt - fb * _FLUSH
                pltpu.sync_copy(
                    bsrc_hbm.at[kb, pl.ds(fb * _FLUSH, _FLUSH)], psrc)
                pltpu.sync_copy(
                    bdl_hbm.at[kb, pl.ds(fb * _FLUSH, _FLUSH)], pdst)
                for w in range(_NBUF - 1):

                    @pl.when(jnp.any(rem > w * _G))
                    def _():
                        gather(w, w)

                def pcond(pi):
                    return jnp.any(rem > pi * _NBUF * _G) & (pi < _NCH // _NBUF)

                def grp(pi):
                    for b in range(_NBUF):
                        g = pi * _NBUF + b

                        @pl.when(jnp.any(rem > g * _G))
                        def _():
                            pltpu.make_async_copy(
                                a_hbm.at[psrc.at[pl.ds(g * _G, _G)]],
                                rows.at[b], sem).wait()
                            nxt = g + _NBUF - 1

                            @pl.when(jnp.any(rem > nxt * _G)
                                     & (nxt < _NCH))
                            def _():
                                gather(nxt, (b + _NBUF - 1) % _NBUF)

                            accum(g, b)
                    return pi + 1

                lax.while_loop(pcond, grp, jnp.int32(0))
                return fb + 1

            lax.while_loop(cond, body, jnp.int32(0))
            pltpu.sync_copy(acc.at[pl.ds(0, _HALF)], seg_hbm.at[pl.ds(lo, _HALF)])

    return k(a, bsrc, bdl, counts)


def kernel(x, xyz, params, edge_index):
    src = edge_index[0]
    dst = edge_index[1]
    bsrc, bdl, counts = _bucketize(src, dst)

    h = x
    for i in range(2):
        theta = params[f"theta_w{i}"]
        phi = params[f"phi_w{i}"]
        w2 = jnp.concatenate([theta, phi - theta], axis=1)
        bias2 = jnp.concatenate(
            [jnp.zeros((D,), jnp.float32),
             params[f"theta_b{i}"] + params[f"phi_b{i}"]])
        a, bvec = _mm_ab(h, w2, bias2)
        seg = _segmax(a, bsrc, bdl, counts)
        u, s, q = _combine(seg, bvec, h)
        h = _bn(u, s, q, params[f"bn_g{i}"], params[f"bn_b{i}"])

    z, s, q = _mm(h, params["l1_w"], params["l1_b"], relu=True, stats=True)
    h = _bn(z, s, q, params["g1"], params["be1"])
    z, s, q = _mm(h, params["l2_w"], params["l2_b"], relu=True, stats=True)
    h = _bn(z, s, q, params["g2"], params["be2"])
    z, s, q = _mm(h, params["l3_w"], params["l3_b"], relu=True, stats=True)
    h = _bn(z, s, q, params["g3"], params["be3"])
    z = _mm(h, params["l4_w"], params["l4_b"])[0]

    out = z.reshape(B, -1, 256).transpose(0, 2, 1)
    return (out, xyz.reshape(B, -1, 3))


# speedup vs baseline: 1.0531x; 1.0531x over previous
import functools

import jax
import jax.numpy as jnp
from jax import lax
from jax.experimental import pallas as pl
from jax.experimental.pallas import tpu as pltpu
from jax.experimental.pallas import tpu_sc as plsc

N = 16384
E = 262144
D = 256
B = 32
EPS = 1e-5

_SENT = -3.0e38
_BM = 512


def _mm_body(x_ref, w_ref, b_ref, *out_refs, relu, stats):
    y = jnp.dot(x_ref[...], w_ref[...], preferred_element_type=jnp.float32,
                precision=lax.Precision.HIGHEST)
    y = y + b_ref[...]
    if relu:
        y = jnp.maximum(y, 0.0)
    out_refs[0][...] = y
    if stats:
        s_ref, q_ref = out_refs[1], out_refs[2]

        @pl.when(pl.program_id(0) == 0)
        def _():
            s_ref[...] = jnp.zeros_like(s_ref)
            q_ref[...] = jnp.zeros_like(q_ref)

        s_ref[...] += jnp.sum(y, axis=0, keepdims=True)
        q_ref[...] += jnp.sum(y * y, axis=0, keepdims=True)


def _mm(x, w, bias, relu=False, stats=False):
    n, di = x.shape
    do = w.shape[1]
    outs = [jax.ShapeDtypeStruct((n, do), jnp.float32)]
    out_specs = [pl.BlockSpec((_BM, do), lambda i: (i, 0))]
    if stats:
        outs += [jax.ShapeDtypeStruct((1, do), jnp.float32)] * 2
        out_specs += [pl.BlockSpec((1, do), lambda i: (0, 0))] * 2
    return pl.pallas_call(
        functools.partial(_mm_body, relu=relu, stats=stats),
        grid=(n // _BM,),
        in_specs=[
            pl.BlockSpec((_BM, di), lambda i: (i, 0)),
            pl.BlockSpec((di, do), lambda i: (0, 0)),
            pl.BlockSpec((1, do), lambda i: (0, 0)),
        ],
        out_specs=out_specs,
        out_shape=outs,
    )(x, w, bias.reshape(1, -1))


def _mm_ab_body(x_ref, w_ref, b_ref, a_ref, bo_ref):
    y = jnp.dot(x_ref[...], w_ref[...], preferred_element_type=jnp.float32,
                precision=lax.Precision.HIGHEST)
    y = y + b_ref[...]
    a_ref[...] = y[:, :D]
    bo_ref[...] = y[:, D:]


def _mm_ab(x, w2, bias2):
    n = x.shape[0]
    return pl.pallas_call(
        _mm_ab_body,
        grid=(n // _BM,),
        in_specs=[
            pl.BlockSpec((_BM, D), lambda i: (i, 0)),
            pl.BlockSpec((D, 2 * D), lambda i: (0, 0)),
            pl.BlockSpec((1, 2 * D), lambda i: (0, 0)),
        ],
        out_specs=[
            pl.BlockSpec((_BM, D), lambda i: (i, 0)),
            pl.BlockSpec((_BM, D), lambda i: (i, 0)),
        ],
        out_shape=[jax.ShapeDtypeStruct((n, D), jnp.float32)] * 2,
    )(x, w2, bias2.reshape(1, -1))


def _combine_body(seg_ref, bv_ref, h_ref, u_ref, s_ref, q_ref):
    seg = seg_ref[...]
    agg = jnp.where(seg < -1e38, 0.0, seg + bv_ref[...])
    u = jnp.maximum(agg + h_ref[...], 0.0)
    u_ref[...] = u

    @pl.when(pl.program_id(0) == 0)
    def _():
        s_ref[...] = jnp.zeros_like(s_ref)
        q_ref[...] = jnp.zeros_like(q_ref)

    s_ref[...] += jnp.sum(u, axis=0, keepdims=True)
    q_ref[...] += jnp.sum(u * u, axis=0, keepdims=True)


def _combine(seg, bvec, h):
    n = h.shape[0]
    return pl.pallas_call(
        _combine_body,
        grid=(n // _BM,),
        in_specs=[pl.BlockSpec((_BM, D), lambda i: (i, 0))] * 3,
        out_specs=[
            pl.BlockSpec((_BM, D), lambda i: (i, 0)),
            pl.BlockSpec((1, D), lambda i: (0, 0)),
            pl.BlockSpec((1, D), lambda i: (0, 0)),
        ],
        out_shape=[
            jax.ShapeDtypeStruct((n, D), jnp.float32),
            jax.ShapeDtypeStruct((1, D), jnp.float32),
            jax.ShapeDtypeStruct((1, D), jnp.float32),
        ],
    )(seg, bvec, h)


def _bn_mm_body(y_ref, s_ref, q_ref, g_ref, b_ref, w_ref, bias_ref,
                *out_refs, relu, stats, split, n_rows):
    mu = s_ref[...] / n_rows
    var = q_ref[...] / n_rows - mu * mu
    sc = g_ref[...] * lax.rsqrt(var + EPS)
    sh = b_ref[...] - mu * sc
    w = w_ref[...]
    ws = w * sc
    shw = jnp.sum(w * sh, axis=0, keepdims=True)
    z = jnp.dot(y_ref[...], ws, preferred_element_type=jnp.float32,
                precision=lax.Precision.HIGHEST)
    z = z + shw + bias_ref[...]
    if relu:
        z = jnp.maximum(z, 0.0)
    if split:
        out_refs[0][...] = z[:, :D]
        out_refs[1][...] = z[:, D:]
        return
    out_refs[0][...] = z
    if stats:
        st_ref, qt_ref = out_refs[1], out_refs[2]

        @pl.when(pl.program_id(0) == 0)
        def _():
            st_ref[...] = jnp.zeros_like(st_ref)
            qt_ref[...] = jnp.zeros_like(qt_ref)

        st_ref[...] += jnp.sum(z, axis=0, keepdims=True)
        qt_ref[...] += jnp.sum(z * z, axis=0, keepdims=True)


def _bn_mm(y, s, q, g, b, w, bias, relu=False, stats=False, split=False):
    n, di = y.shape
    do = w.shape[1]
    if split:
        outs = [jax.ShapeDtypeStruct((n, D), jnp.float32)] * 2
        out_specs = [pl.BlockSpec((_BM, D), lambda i: (i, 0))] * 2
    else:
        outs = [jax.ShapeDtypeStruct((n, do), jnp.float32)]
        out_specs = [pl.BlockSpec((_BM, do), lambda i: (i, 0))]
        if stats:
            outs += [jax.ShapeDtypeStruct((1, do), jnp.float32)] * 2
            out_specs += [pl.BlockSpec((1, do), lambda i: (0, 0))] * 2
    return pl.pallas_call(
        functools.partial(_bn_mm_body, relu=relu, stats=stats, split=split,
                          n_rows=float(n)),
        grid=(n // _BM,),
        in_specs=[
            pl.BlockSpec((_BM, di), lambda i: (i, 0)),
            pl.BlockSpec((di, 1), lambda i: (0, 0)),
            pl.BlockSpec((di, 1), lambda i: (0, 0)),
            pl.BlockSpec((di, 1), lambda i: (0, 0)),
            pl.BlockSpec((di, 1), lambda i: (0, 0)),
            pl.BlockSpec((di, do), lambda i: (0, 0)),
            pl.BlockSpec((1, do), lambda i: (0, 0)),
        ],
        out_specs=out_specs,
        out_shape=outs,
    )(y, s.reshape(-1, 1), q.reshape(-1, 1), g.reshape(-1, 1),
      b.reshape(-1, 1), w, bias.reshape(1, -1))


def _combine2_body(seg_ref, bv_ref, y_ref, s_ref, q_ref, g_ref, b_ref,
                   u_ref, st_ref, qt_ref, *, n_rows):
    mu = s_ref[...] / n_rows
    var = q_ref[...] / n_rows - mu * mu
    sc = g_ref[...] * lax.rsqrt(var + EPS)
    sh = b_ref[...] - mu * sc
    h = y_ref[...] * sc + sh
    seg = seg_ref[...]
    agg = jnp.where(seg < -1e38, 0.0, seg + bv_ref[...])
    u = jnp.maximum(agg + h, 0.0)
    u_ref[...] = u

    @pl.when(pl.program_id(0) == 0)
    def _():
        st_ref[...] = jnp.zeros_like(st_ref)
        qt_ref[...] = jnp.zeros_like(qt_ref)

    st_ref[...] += jnp.sum(u, axis=0, keepdims=True)
    qt_ref[...] += jnp.sum(u * u, axis=0, keepdims=True)


def _combine2(seg, bvec, y, s, q, g, b):
    n = y.shape[0]
    return pl.pallas_call(
        functools.partial(_combine2_body, n_rows=float(n)),
        grid=(n // _BM,),
        in_specs=[pl.BlockSpec((_BM, D), lambda i: (i, 0))] * 3
        + [pl.BlockSpec((1, D), lambda i: (0, 0))] * 4,
        out_specs=[
            pl.BlockSpec((_BM, D), lambda i: (i, 0)),
            pl.BlockSpec((1, D), lambda i: (0, 0)),
            pl.BlockSpec((1, D), lambda i: (0, 0)),
        ],
        out_shape=[
            jax.ShapeDtypeStruct((n, D), jnp.float32),
            jax.ShapeDtypeStruct((1, D), jnp.float32),
            jax.ShapeDtypeStruct((1, D), jnp.float32),
        ],
    )(seg, bvec, y, s.reshape(1, -1), q.reshape(1, -1), g.reshape(1, -1),
      b.reshape(1, -1))


def _bn_body(y_ref, s_ref, q_ref, g_ref, b_ref, o_ref, *, n_rows):
    mu = s_ref[...] / n_rows
    var = q_ref[...] / n_rows - mu * mu
    sc = g_ref[...] * lax.rsqrt(var + EPS)
    sh = b_ref[...] - mu * sc
    o_ref[...] = y_ref[...] * sc + sh


def _bn(y, s, q, g, b):
    n, do = y.shape
    return pl.pallas_call(
        functools.partial(_bn_body, n_rows=float(n)),
        grid=(n // _BM,),
        in_specs=[
            pl.BlockSpec((_BM, do), lambda i: (i, 0)),
            pl.BlockSpec((1, do), lambda i: (0, 0)),
            pl.BlockSpec((1, do), lambda i: (0, 0)),
            pl.BlockSpec((1, do), lambda i: (0, 0)),
            pl.BlockSpec((1, do), lambda i: (0, 0)),
        ],
        out_specs=pl.BlockSpec((_BM, do), lambda i: (i, 0)),
        out_shape=jax.ShapeDtypeStruct((n, do), jnp.float32),
    )(y, s.reshape(1, -1), q.reshape(1, -1), g.reshape(1, -1), b.reshape(1, -1))


_NW = 32
_RANGE = N // _NW
_HALF = _RANGE // 2
_NB = 2 * _NW
_SCAN = 8192
_GRAN = 1024
_FLUSH = 2048
_RING = 4096
_G = 32
_NCH = _FLUSH // _G
_NBUF = 4


def _bucketize(src, dst):
    mesh = plsc.VectorSubcoreMesh(core_axis_name="c", subcore_axis_name="s")

    @functools.partial(
        pl.kernel,
        mesh=mesh,
        compiler_params=pltpu.CompilerParams(needs_layout_passes=False),
        out_type=(
            jax.ShapeDtypeStruct((_NB, E), jnp.int32),
            jax.ShapeDtypeStruct((_NB, E), jnp.int32),
            jax.ShapeDtypeStruct((_NB, 16), jnp.int32),
        ),
        scratch_types=[
            pltpu.VMEM((_SCAN,), jnp.int32),
            pltpu.VMEM((_SCAN,), jnp.int32),
            pltpu.VMEM((_RING,), jnp.int32),
            pltpu.VMEM((_RING,), jnp.int32),
            pltpu.VMEM((_RING,), jnp.int32),
            pltpu.VMEM((_RING,), jnp.int32),
            pltpu.VMEM((16,), jnp.int32),
        ],
    )
    def k(src_hbm, dst_hbm, bsrc_hbm, bdl_hbm, cnt_hbm,
          sbuf, dbuf, rs0, rd0, rs1, rd1, cntv):
        wid = lax.axis_index("s") * 2 + lax.axis_index("c")
        kb0 = wid * 2
        lo = wid * _RANGE

        def prefill(i, _):
            z = jnp.zeros((16,), jnp.int32)
            hv = jnp.full((16,), _HALF, jnp.int32)
            rs0[pl.ds(i * 16, 16)] = z
            rs1[pl.ds(i * 16, 16)] = z
            rd0[pl.ds(i * 16, 16)] = hv
            rd1[pl.ds(i * 16, 16)] = hv
            return 0

        lax.fori_loop(0, _RING // 16, prefill, 0)

        def flush(f, pend, rs, rd, kb, final):
            def cond(f):
                lim = (f + 1) * _FLUSH if not final else f * _FLUSH
                return jnp.any(pend > lim - (0 if final else 1))

            def body(f):
                blk = lax.rem(f, 2) * _FLUSH
                pltpu.sync_copy(rs.at[pl.ds(blk, _FLUSH)],
                                bsrc_hbm.at[kb, pl.ds(f * _FLUSH, _FLUSH)])
                pltpu.sync_copy(rd.at[pl.ds(blk, _FLUSH)],
                                bdl_hbm.at[kb, pl.ds(f * _FLUSH, _FLUSH)])
                return f + 1

            return lax.while_loop(cond, body, f)

        def outer(oc, carry):
            base = oc * _SCAN
            pltpu.sync_copy(src_hbm.at[pl.ds(base, _SCAN)], sbuf)
            pltpu.sync_copy(dst_hbm.at[pl.ds(base, _SCAN)], dbuf)

            def gran(gi, carry):
                f0, f1, p0, p1 = carry

                def scan16(j, carry):
                    p0, p1 = carry
                    off = gi * _GRAN + j * 16
                    sv = sbuf[pl.ds(off, 16)]
                    dv = dbuf[pl.ds(off, 16)]
                    dl = dv - lo
                    m0 = (dl >= 0) & (dl < _HALF)
                    m1 = (dl >= _HALF) & (dl < _RANGE)
                    cs0 = plsc.cumsum(m0.astype(jnp.int32))
                    pos0 = (p0 + cs0 - 1) & (_RING - 1)
                    plsc.store_scatter(rs0, [pos0], sv, mask=m0)
                    plsc.store_scatter(rd0, [pos0], dl, mask=m0)
                    cs1 = plsc.cumsum(m1.astype(jnp.int32))
                    pos1 = (p1 + cs1 - 1) & (_RING - 1)
                    plsc.store_scatter(rs1, [pos1], sv, mask=m1)
                    plsc.store_scatter(rd1, [pos1], dl - _HALF, mask=m1)
                    return (p0 + plsc.all_reduce_population_count(m0),
                            p1 + plsc.all_reduce_population_count(m1))

                p0, p1 = lax.fori_loop(0, _GRAN // 16, scan16, (p0, p1))
                f0 = flush(f0, p0, rs0, rd0, kb0, False)
                f1 = flush(f1, p1, rs1, rd1, kb0 + 1, False)
                return (f0, f1, p0, p1)

            return lax.fori_loop(0, _SCAN // _GRAN, gran, carry)

        z16 = jnp.zeros((16,), jnp.int32)
        f0, f1, p0, p1 = lax.fori_loop(
            0, E // _SCAN, outer, (jnp.int32(0), jnp.int32(0), z16, z16))
        f0 = flush(f0, p0, rs0, rd0, kb0, True)
        f1 = flush(f1, p1, rs1, rd1, kb0 + 1, True)
        cntv[pl.ds(0, 16)] = p0
        pltpu.sync_copy(cntv, cnt_hbm.at[kb0])
        cntv[pl.ds(0, 16)] = p1
        pltpu.sync_copy(cntv, cnt_hbm.at[kb0 + 1])

    return k(src, dst)


def _segmax(a, bsrc, bdl, counts):
    mesh = plsc.VectorSubcoreMesh(core_axis_name="c", subcore_axis_name="s")

    @functools.partial(
        pl.kernel,
        mesh=mesh,
        compiler_params=pltpu.CompilerParams(needs_layout_passes=False),
        out_type=jax.ShapeDtypeStruct((N, D), jnp.float32),
        scratch_types=[
            pltpu.VMEM((_HALF + 1, D), jnp.float32),
            pltpu.VMEM((_FLUSH,), jnp.int32),
            pltpu.VMEM((_FLUSH,), jnp.int32),
            pltpu.VMEM((_NBUF, _G, D), jnp.float32),
            pltpu.VMEM((16,), jnp.int32),
            pltpu.SemaphoreType.DMA,
        ],
    )
    def k(a_hbm, bsrc_hbm, bdl_hbm, cnt_hbm, seg_hbm,
          acc, psrc, pdst, rows, cntv, sem):
        wid = lax.axis_index("s") * 2 + lax.axis_index("c")
        lanes = jnp.arange(16, dtype=jnp.int32)

        for bi in range(2):
            kb = wid * 2 + bi
            lo = wid * _RANGE + bi * _HALF

            def initrow(i, _):
                for c in range(D // 16):
                    acc[i, pl.ds(c * 16, 16)] = jnp.full((16,), _SENT, jnp.float32)
                return 0

            lax.fori_loop(0, _HALF + 1, initrow, 0)

            pltpu.sync_copy(cnt_hbm.at[kb], cntv)
            cnt = cntv[pl.ds(0, 16)]

            def gather(g, buf):
                return pltpu.async_copy(
                    a_hbm.at[psrc.at[pl.ds(g * _G, _G)]], rows.at[buf], sem)

            def accum(g, buf):
                def row1(j):
                    jh = (j // 16) * 16
                    jm = j - jh
                    dchunk = pdst[pl.ds(g * _G + jh, 16)]
                    jvec = jnp.broadcast_to(jm, (16, 1)).astype(jnp.int32)
                    dlb = lax.gather(
                        dchunk, jvec,
                        lax.GatherDimensionNumbers(
                            offset_dims=(), collapsed_slice_dims=(0,),
                            start_index_map=(0,)),
                        (1,), mode=lax.GatherScatterMode.PROMISE_IN_BOUNDS)
                    for c in range(D // 16):
                        colidx = c * 16 + lanes
                        gv = rows[buf, j, pl.ds(c * 16, 16)]
                        av = plsc.load_gather(acc, [dlb, colidx])
                        plsc.store_scatter(
                            acc, [dlb, colidx], jnp.maximum(av, gv))

                def rowj(i, _):
                    for u in range(4):
                        row1(i * 4 + u)
                    return 0

                lax.fori_loop(0, _G // 4, rowj, 0)

            def cond(fb):
                return jnp.any(cnt > fb * _FLUSH)

            def body(fb):
                rem = cnt - fb * _FLUSH
                pltpu.sync_copy(
                    bsrc_hbm.at[kb, pl.ds(fb * _FLUSH, _FLUSH)], psrc)
                pltpu.sync_copy(
                    bdl_hbm.at[kb, pl.ds(fb * _FLUSH, _FLUSH)], pdst)
                for w in range(_NBUF - 1):

                    @pl.when(jnp.any(rem > w * _G))
                    def _():
                        gather(w, w)

                def pcond(pi):
                    return jnp.any(rem > pi * _NBUF * _G) & (pi < _NCH // _NBUF)

                def grp(pi):
                    for b in range(_NBUF):
                        g = pi * _NBUF + b

                        @pl.when(jnp.any(rem > g * _G))
                        def _():
                            pltpu.make_async_copy(
                                a_hbm.at[psrc.at[pl.ds(g * _G, _G)]],
                                rows.at[b], sem).wait()
                            nxt = g + _NBUF - 1

                            @pl.when(jnp.any(rem > nxt * _G)
                                     & (nxt < _NCH))
                            def _():
                                gather(nxt, (b + _NBUF - 1) % _NBUF)

                            accum(g, b)
                    return pi + 1

                lax.while_loop(pcond, grp, jnp.int32(0))
                return fb + 1

            lax.while_loop(cond, body, jnp.int32(0))
            pltpu.sync_copy(acc.at[pl.ds(0, _HALF)], seg_hbm.at[pl.ds(lo, _HALF)])

    return k(a, bsrc, bdl, counts)


def kernel(x, xyz, params, edge_index):
    src = edge_index[0]
    dst = edge_index[1]
    bsrc, bdl, counts = _bucketize(src, dst)

    def ec_weights(i):
        theta = params[f"theta_w{i}"]
        phi = params[f"phi_w{i}"]
        w2 = jnp.concatenate([theta, phi - theta], axis=1)
        bias2 = jnp.concatenate(
            [jnp.zeros((D,), jnp.float32),
             params[f"theta_b{i}"] + params[f"phi_b{i}"]])
        return w2, bias2

    w2, bias2 = ec_weights(0)
    a, bvec = _mm_ab(x, w2, bias2)
    seg = _segmax(a, bsrc, bdl, counts)
    u1, s1, q1 = _combine(seg, bvec, x)

    w2, bias2 = ec_weights(1)
    a, bvec = _bn_mm(u1, s1, q1, params["bn_g0"], params["bn_b0"],
                     w2, bias2, split=True)
    seg = _segmax(a, bsrc, bdl, counts)
    u2, s2, q2 = _combine2(seg, bvec, u1, s1, q1,
                           params["bn_g0"], params["bn_b0"])

    z, s, q = _bn_mm(u2, s2, q2, params["bn_g1"], params["bn_b1"],
                     params["l1_w"], params["l1_b"], relu=True, stats=True)
    z, s, q = _bn_mm(z, s, q, params["g1"], params["be1"],
                     params["l2_w"], params["l2_b"], relu=True, stats=True)
    z, s, q = _bn_mm(z, s, q, params["g2"], params["be2"],
                     params["l3_w"], params["l3_b"], relu=True, stats=True)
    z = _bn_mm(z, s, q, params["g3"], params["be3"],
               params["l4_w"], params["l4_b"])[0]

    out = z.reshape(B, -1, 256).transpose(0, 2, 1)
    return (out, xyz.reshape(B, -1, 3))
